# baseline (speedup 1.0000x reference)
import functools

import jax
import jax.numpy as jnp
from jax import lax
from jax.experimental import pallas as pl
from jax.experimental.pallas import tpu as pltpu
from jax.experimental.pallas import tpu_sc as plsc

_NUM_CORES = 2
_NUM_SUBCORES = 16
_NW = _NUM_CORES * _NUM_SUBCORES
_GROUP = 2


def _gather_kernel(b, t, d):
    rows_per_w = b // _NW
    n_groups = rows_per_w // _GROUP
    mesh = plsc.VectorSubcoreMesh(
        core_axis_name="c",
        subcore_axis_name="s",
        num_cores=_NUM_CORES,
        num_subcores=_NUM_SUBCORES,
    )

    @functools.partial(
        pl.kernel,
        out_type=jax.ShapeDtypeStruct((b, t, d), jnp.float32),
        mesh=mesh,
        scratch_types=[
            pltpu.VMEM((rows_per_w, t), jnp.int32),
            pltpu.VMEM((_GROUP * 2, t, d), jnp.float32),
            pltpu.SemaphoreType.DMA,
            pltpu.SemaphoreType.DMA,
            pltpu.SemaphoreType.DMA,
            pltpu.SemaphoreType.DMA,
        ],
        compiler_params=pltpu.CompilerParams(use_tc_tiling_on_sc=False),
    )
    def k(idx_hbm, table_hbm, out_hbm, idx_v, rows_v, sg0, sg1, sw0, sw1):
        wid = lax.axis_index("s") * _NUM_CORES + lax.axis_index("c")
        base = wid * rows_per_w
        pltpu.sync_copy(idx_hbm.at[pl.ds(base, rows_per_w), :], idx_v)
        sg = (sg0, sg1)
        sw = (sw0, sw1)

        def start_gathers(grp, p):
            for q in range(_GROUP):
                r = grp * _GROUP + q
                pltpu.async_copy(
                    table_hbm.at[idx_v.at[r]],
                    rows_v.at[_GROUP * p + q],
                    sg[p],
                )

        def wait_gathers(p):
            for q in range(_GROUP):
                pltpu.make_async_copy(
                    table_hbm.at[idx_v.at[0]],
                    rows_v.at[_GROUP * p + q],
                    sg[p],
                ).wait()

        def start_writes(grp, p):
            for q in range(_GROUP):
                r = grp * _GROUP + q
                pltpu.async_copy(
                    rows_v.at[_GROUP * p + q],
                    out_hbm.at[base + r],
                    sw[p],
                )

        def wait_writes(p):
            for q in range(_GROUP):
                pltpu.make_async_copy(
                    rows_v.at[_GROUP * p + q],
                    out_hbm.at[base],
                    sw[p],
                ).wait()

        def run_group(grp, p, wait_prev_writes, start_next):
            wait_gathers(p)
            if wait_prev_writes:
                wait_writes(1 - p)
            if start_next:
                start_gathers(grp + 1, 1 - p)
            start_writes(grp, p)

        start_gathers(0, 0)
        run_group(0, 0, False, True)
        run_group(1, 1, True, True)

        def body(jj, carry):
            run_group(2 * jj, 0, True, True)
            run_group(2 * jj + 1, 1, True, True)
            return carry

        lax.fori_loop(1, n_groups // 2 - 1, body, 0)

        run_group(n_groups - 2, 0, True, True)
        run_group(n_groups - 1, 1, True, False)
        wait_writes(1)

    return k


def kernel(idx, table):
    b, t = idx.shape
    v, d = table.shape
    return _gather_kernel(b, t, d)(idx, table)


# speedup vs baseline: 1.2251x; 1.2251x over previous
import functools

import jax
import jax.numpy as jnp
from jax import lax
from jax.experimental import pallas as pl
from jax.experimental.pallas import tpu as pltpu
from jax.experimental.pallas import tpu_sc as plsc

_NUM_CORES = 2
_NUM_SUBCORES = 16
_NW = _NUM_CORES * _NUM_SUBCORES
_CHUNK = 200
_GROUP = 2


def _gather_kernel(n, dp):
    n_per_w = n // _NW
    n_chunks = n_per_w // _CHUNK
    n_groups = n_chunks // _GROUP
    mesh = plsc.VectorSubcoreMesh(
        core_axis_name="c",
        subcore_axis_name="s",
        num_cores=_NUM_CORES,
        num_subcores=_NUM_SUBCORES,
    )

    @functools.partial(
        pl.kernel,
        out_type=jax.ShapeDtypeStruct((n, dp), jnp.float32),
        mesh=mesh,
        scratch_types=[
            pltpu.VMEM((n_per_w,), jnp.int32),
            pltpu.VMEM((_GROUP * 2, _CHUNK, dp), jnp.float32),
            pltpu.SemaphoreType.DMA,
            pltpu.SemaphoreType.DMA,
            pltpu.SemaphoreType.DMA,
            pltpu.SemaphoreType.DMA,
        ],
        compiler_params=pltpu.CompilerParams(use_tc_tiling_on_sc=True),
    )
    def k(idx_hbm, table_hbm, out_hbm, idx_v, rows_v, sg0, sg1, sw0, sw1):
        wid = lax.axis_index("s") * _NUM_CORES + lax.axis_index("c")
        base = wid * n_per_w
        pltpu.sync_copy(idx_hbm.at[pl.ds(base, n_per_w)], idx_v)
        sg = (sg0, sg1)
        sw = (sw0, sw1)

        def start_gathers(grp, p):
            for q in range(_GROUP):
                off = (grp * _GROUP + q) * _CHUNK
                pltpu.async_copy(
                    table_hbm.at[idx_v.at[pl.ds(off, _CHUNK)]],
                    rows_v.at[_GROUP * p + q],
                    sg[p],
                )

        def wait_gathers(p):
            for q in range(_GROUP):
                pltpu.make_async_copy(
                    table_hbm.at[idx_v.at[pl.ds(0, _CHUNK)]],
                    rows_v.at[_GROUP * p + q],
                    sg[p],
                ).wait()

        def start_writes(grp, p):
            for q in range(_GROUP):
                off = (grp * _GROUP + q) * _CHUNK
                pltpu.async_copy(
                    rows_v.at[_GROUP * p + q],
                    out_hbm.at[pl.ds(base + off, _CHUNK)],
                    sw[p],
                )

        def wait_writes(p):
            for q in range(_GROUP):
                pltpu.make_async_copy(
                    rows_v.at[_GROUP * p + q],
                    out_hbm.at[pl.ds(base, _CHUNK)],
                    sw[p],
                ).wait()

        def run_group(grp, p, wait_prev_writes, start_next):
            wait_gathers(p)
            if wait_prev_writes:
                wait_writes(1 - p)
            if start_next:
                start_gathers(grp + 1, 1 - p)
            start_writes(grp, p)

        start_gathers(0, 0)
        run_group(0, 0, False, True)
        run_group(1, 1, True, True)

        def body(jj, carry):
            run_group(2 * jj, 0, True, True)
            run_group(2 * jj + 1, 1, True, True)
            return carry

        lax.fori_loop(1, n_groups // 2 - 1, body, 0)

        run_group(n_groups - 2, 0, True, True)
        run_group(n_groups - 1, 1, True, False)
        wait_writes(1)

    return k


def kernel(idx, table):
    b, t = idx.shape
    v, d = table.shape
    n = b * t
    table_p = jnp.pad(table, ((0, 0), (0, 128 - d)))
    out = _gather_kernel(n, 128)(idx.reshape(n), table_p)
    return out[:, :d].reshape(b, t, d)
